# baseline (speedup 1.0000x reference)
import functools

import jax
import jax.numpy as jnp
from jax import lax
from jax.experimental import pallas as pl
from jax.experimental.pallas import tpu as pltpu
from jax.experimental.pallas import tpu_sc as plsc

D_MODEL = 64
SCALE = 8.0
CHUNK = 64
DEPTH = 4
NRING = 2 * DEPTH


@functools.lru_cache(maxsize=None)
def _make_kernel(B):
    info = plsc.get_sparse_core_info()
    nc, ns = info.num_cores, info.num_subcores
    nw = nc * ns
    b_per_w = B // nw
    n_chunks = b_per_w // CHUNK
    n_outer = n_chunks // NRING
    assert b_per_w * nw == B and n_outer * NRING == n_chunks

    mesh = plsc.VectorSubcoreMesh(core_axis_name="c", subcore_axis_name="s")

    @functools.partial(
        pl.kernel,
        mesh=mesh,
        out_type=jax.ShapeDtypeStruct((B, 128), jnp.float32),
        compiler_params=pltpu.CompilerParams(
            use_tc_tiling_on_sc=True, needs_layout_passes=False
        ),
        scratch_types=(
            [pltpu.VMEM((b_per_w,), jnp.int32)]
            + [pltpu.VMEM((CHUNK, 128), jnp.float32) for _ in range(NRING)]
            + [pltpu.SemaphoreType.DMA for _ in range(NRING)]
            + [pltpu.SemaphoreType.DMA for _ in range(NRING)]
        ),
    )
    def emb_kernel(x_hbm, lut_hbm, out_hbm, idx_v, *rest):
        gbufs = rest[:NRING]
        gsems = rest[NRING:2 * NRING]
        osems = rest[2 * NRING:]
        wid = lax.axis_index("s") * nc + lax.axis_index("c")
        base = wid * b_per_w

        pltpu.sync_copy(x_hbm.at[pl.ds(base, b_per_w)], idx_v)

        def gather_desc(g, b):
            idx_slice = idx_v.at[pl.ds(pl.multiple_of(g * CHUNK, CHUNK), CHUNK)]
            return pltpu.make_async_copy(lut_hbm.at[idx_slice], gbufs[b], gsems[b])

        def write_desc(g, b):
            dst = out_hbm.at[pl.ds(base + pl.multiple_of(g * CHUNK, CHUNK), CHUNK)]
            return pltpu.make_async_copy(gbufs[b], dst, osems[b])

        for b in range(DEPTH):
            gather_desc(b, b).start()

        def outer(o, carry):
            g0 = o * NRING
            for s in range(NRING):
                g = g0 + s
                gather_desc(g, s).wait()
                write_desc(g, s).start()

                s2 = (s + DEPTH) % NRING

                @pl.when(g >= DEPTH)
                def _():
                    write_desc(g - DEPTH, s2).wait()

                @pl.when(g + DEPTH < n_chunks)
                def _():
                    gather_desc(g + DEPTH, s2).start()
            return carry

        lax.fori_loop(0, n_outer, outer, 0)

        for g in range(n_chunks - DEPTH, n_chunks):
            write_desc(g, g % NRING).wait()

    return emb_kernel


def kernel(x, lut):
    NB, NT = x.shape
    B = NB * NT
    xf = x.reshape(B).astype(jnp.int32)
    lutp = lut.reshape(-1, 2 * D_MODEL)
    relay = _make_kernel(B)(xf >> 1, lutp)
    parity = (xf & 1).astype(bool)
    out = jnp.where(parity[:, None], relay[:, D_MODEL:], relay[:, :D_MODEL])
    return (out * SCALE).reshape(NB, NT, D_MODEL)


# speedup vs baseline: 1.5434x; 1.5434x over previous
import functools

import jax
import jax.numpy as jnp
from jax import lax
from jax.experimental import pallas as pl
from jax.experimental.pallas import tpu as pltpu
from jax.experimental.pallas import tpu_sc as plsc

D_MODEL = 64
SCALE = 8.0
CHUNK = 64
DEPTH = 4
NRING = 2 * DEPTH


@functools.lru_cache(maxsize=None)
def _make_kernel(B):
    info = plsc.get_sparse_core_info()
    nc, ns = info.num_cores, info.num_subcores
    nw = nc * ns
    b_per_w = B // nw
    n_chunks = b_per_w // CHUNK
    n_outer = n_chunks // NRING
    assert b_per_w * nw == B and n_outer * NRING == n_chunks

    mesh = plsc.VectorSubcoreMesh(core_axis_name="c", subcore_axis_name="s")

    @functools.partial(
        pl.kernel,
        mesh=mesh,
        out_type=jax.ShapeDtypeStruct((B, 128), jnp.float32),
        compiler_params=pltpu.CompilerParams(
            use_tc_tiling_on_sc=True, needs_layout_passes=False
        ),
        scratch_types=(
            [pltpu.VMEM((b_per_w,), jnp.int32)]
            + [pltpu.VMEM((CHUNK, 128), jnp.float32) for _ in range(NRING)]
            + [pltpu.SemaphoreType.DMA for _ in range(NRING)]
            + [pltpu.SemaphoreType.DMA for _ in range(NRING)]
        ),
    )
    def emb_kernel(x_hbm, lut_hbm, out_hbm, idx_v, *rest):
        gbufs = rest[:NRING]
        gsems = rest[NRING:2 * NRING]
        osems = rest[2 * NRING:]
        wid = lax.axis_index("s") * nc + lax.axis_index("c")
        base = wid * b_per_w

        pltpu.sync_copy(x_hbm.at[pl.ds(base, b_per_w)], idx_v)

        def gather_desc(g, b):
            idx_slice = idx_v.at[pl.ds(pl.multiple_of(g * CHUNK, CHUNK), CHUNK)]
            return pltpu.make_async_copy(lut_hbm.at[idx_slice], gbufs[b], gsems[b])

        def write_desc(g, b):
            dst = out_hbm.at[pl.ds(base + pl.multiple_of(g * CHUNK, CHUNK), CHUNK)]
            return pltpu.make_async_copy(gbufs[b], dst, osems[b])

        for b in range(DEPTH):
            gather_desc(b, b).start()

        def outer(o, carry):
            g0 = o * NRING
            for s in range(NRING):
                g = g0 + s
                gather_desc(g, s).wait()
                write_desc(g, s).start()

                s2 = (s + DEPTH) % NRING

                @pl.when(g >= DEPTH)
                def _():
                    write_desc(g - DEPTH, s2).wait()

                @pl.when(g + DEPTH < n_chunks)
                def _():
                    gather_desc(g + DEPTH, s2).start()
            return carry

        lax.fori_loop(0, n_outer, outer, 0)

        for g in range(n_chunks - DEPTH, n_chunks):
            write_desc(g, g % NRING).wait()

    return emb_kernel


def kernel(x, lut):
    NB, NT = x.shape
    B = NB * NT
    xf = x.reshape(B).astype(jnp.int32)
    lutp = jnp.pad(lut, ((0, 0), (0, 128 - lut.shape[1]))) * SCALE
    out = _make_kernel(B)(xf, lutp)
    return out.reshape(NB, NT, 128)[:, :, :D_MODEL]
